# baseline (speedup 1.0000x reference)
import functools

import jax
import jax.numpy as jnp
from jax import lax
from jax.experimental import pallas as pl
from jax.experimental.pallas import tpu as pltpu
from jax.experimental.pallas import tpu_sc as plsc

T = 2048
D = 768
E = 8
F = 3072
TB = 256
BLK = 448
NBLK = (T + BLK - 1) // BLK + E
TPAD = NBLK * BLK
LBW = 0.01
_NC, _NS = 2, 16
_NW = _NC * _NS


def _routing_body(x_ref, wr_ref, eidx_ref, rank_ref, w_ref, plan_ref,
                  aux_ref, cnt, sp):
    i = pl.program_id(0)

    @pl.when(i == 0)
    def _init():
        cnt[...] = jnp.zeros_like(cnt)
        sp[...] = jnp.zeros_like(sp)

    logits = jnp.dot(x_ref[...], wr_ref[...],
                     preferred_element_type=jnp.float32)
    m = jnp.max(logits, axis=-1, keepdims=True)
    ex = jnp.exp(logits - m)
    s = jnp.sum(ex, axis=-1, keepdims=True)
    probs16 = jnp.concatenate(
        [ex / s, jnp.zeros((TB, 16 - E), jnp.float32)], axis=-1)
    lane8 = lax.broadcasted_iota(jnp.int32, (TB, E), 1)
    e = jnp.min(jnp.where(logits == m, lane8, E), axis=-1).astype(jnp.int32)
    onehot = (lax.broadcasted_iota(jnp.int32, (TB, 16), 1)
              == e[:, None]).astype(jnp.float32)
    tri = (lax.broadcasted_iota(jnp.int32, (TB, TB), 0)
           >= lax.broadcasted_iota(jnp.int32, (TB, TB), 1)).astype(jnp.float32)
    csum = jnp.dot(tri, onehot, preferred_element_type=jnp.float32)
    prev = cnt[0, :]
    rank = jnp.sum(onehot * (prev[None, :] + csum), axis=-1) - 1.0
    eidx_ref[...] = e
    rank_ref[...] = rank.astype(jnp.int32)
    w_ref[...] = 1.0 / s[:, 0]
    cnt[0, :] = prev + jnp.sum(onehot, axis=0)
    sp[0, :] = sp[0, :] + jnp.sum(probs16, axis=0)

    @pl.when(i == pl.num_programs(0) - 1)
    def _fin():
        cfin = cnt[0:1, :]
        nb = jnp.floor((cfin + (BLK - 1)) * (1.0 / BLK))
        tri16 = (lax.broadcasted_iota(jnp.int32, (16, 16), 0)
                 <= lax.broadcasted_iota(jnp.int32, (16, 16), 1)
                 ).astype(jnp.float32)
        cumnb = jnp.dot(nb, tri16, preferred_element_type=jnp.float32)
        offs = (cumnb - nb) * BLK
        col = lax.broadcasted_iota(jnp.int32, (16, 16), 1)
        row = lax.broadcasted_iota(jnp.int32, (16, 16), 0)
        ble = jnp.sum(jnp.where(
            jnp.logical_and(col < E,
                            jnp.broadcast_to(cumnb, (16, 16))
                            <= row.astype(jnp.float32)),
            1, 0), axis=-1, keepdims=False)
        ble = jnp.minimum(ble, E - 1)
        nact = jnp.sum(jnp.where(col < E, jnp.broadcast_to(nb, (16, 16)), 0.0),
                       axis=-1)[0:1]
        plan_ref[...] = jnp.concatenate([
            offs.astype(jnp.int32),
            ble.astype(jnp.int32)[None, :],
            jnp.broadcast_to(nact.astype(jnp.int32)[None, :1], (1, 16)),
            jnp.zeros((5, 16), jnp.int32)], axis=0)
        aux = (LBW * E / (T * T)) * jnp.sum(sp[0, :] * cfin[0, :])
        aux_ref[...] = jnp.broadcast_to(aux, (1, 1))


def _routing(x_flat, wr):
    nb = T // TB
    return pl.pallas_call(
        _routing_body,
        grid=(nb,),
        in_specs=[
            pl.BlockSpec((TB, D), lambda i: (i, 0)),
            pl.BlockSpec((D, E), lambda i: (0, 0)),
        ],
        out_specs=[
            pl.BlockSpec((TB,), lambda i: (i,)),
            pl.BlockSpec((TB,), lambda i: (i,)),
            pl.BlockSpec((TB,), lambda i: (i,)),
            pl.BlockSpec((8, 16), lambda i: (0, 0)),
            pl.BlockSpec((1, 1), lambda i: (0, 0)),
        ],
        out_shape=[
            jax.ShapeDtypeStruct((T,), jnp.int32),
            jax.ShapeDtypeStruct((T,), jnp.int32),
            jax.ShapeDtypeStruct((T,), jnp.float32),
            jax.ShapeDtypeStruct((8, 16), jnp.int32),
            jax.ShapeDtypeStruct((1, 1), jnp.float32),
        ],
        scratch_shapes=[pltpu.VMEM((1, 16), jnp.float32),
                        pltpu.VMEM((1, 16), jnp.float32)],
        compiler_params=pltpu.CompilerParams(
            dimension_semantics=("arbitrary",)),
    )(x_flat, wr)


def _sc_dispatch(x_flat, eidx, rank, plan):
    tpw = T // _NW
    mesh = plsc.VectorSubcoreMesh(core_axis_name="c", subcore_axis_name="s")

    @functools.partial(
        pl.kernel, mesh=mesh,
        out_type=[jax.ShapeDtypeStruct((TPAD, D), jnp.float32),
                  jax.ShapeDtypeStruct((T,), jnp.int32)],
        scratch_types=[pltpu.VMEM((16,), jnp.int32),
                       pltpu.VMEM((tpw,), jnp.int32),
                       pltpu.VMEM((tpw,), jnp.int32),
                       pltpu.VMEM((tpw,), jnp.int32),
                       pltpu.VMEM((tpw, D), jnp.float32),
                       pltpu.SemaphoreType.DMA,
                       pltpu.SemaphoreType.DMA],
    )
    def k(x_hbm, e_hbm, r_hbm, p_hbm, xs_hbm, dest_hbm,
          ovm, evm, rvm, dvm, rows, sem, xsem):
        wid = lax.axis_index("s") * _NC + lax.axis_index("c")
        base = wid * tpw
        xcp = pltpu.async_copy(x_hbm.at[pl.ds(base, tpw)], rows, xsem)
        ecp = pltpu.async_copy(e_hbm.at[pl.ds(base, tpw)], evm, sem)
        rcp = pltpu.async_copy(r_hbm.at[pl.ds(base, tpw)], rvm, sem)
        pcp = pltpu.async_copy(p_hbm.at[0], ovm, sem)
        ecp.wait()
        rcp.wait()
        pcp.wait()
        offs_reg = ovm[...]
        for c in range(tpw // 16):
            sl = pl.ds(c * 16, 16)
            dvm[sl] = offs_reg[evm[sl]] + rvm[sl]
        xcp.wait()
        pltpu.async_copy(rows, xs_hbm.at[dvm], sem).wait()
        pltpu.sync_copy(dvm, dest_hbm.at[pl.ds(base, tpw)])

    return k(x_flat, eidx, rank, plan)


def _sc_combine(ys, dest, w):
    tpw = T // _NW
    mesh = plsc.VectorSubcoreMesh(core_axis_name="c", subcore_axis_name="s")

    @functools.partial(
        pl.kernel, mesh=mesh,
        out_type=jax.ShapeDtypeStruct((T, D), jnp.float32),
        scratch_types=[pltpu.VMEM((tpw,), jnp.int32),
                       pltpu.VMEM((tpw, D), jnp.float32),
                       pltpu.VMEM((tpw,), jnp.float32),
                       pltpu.SemaphoreType.DMA,
                       pltpu.SemaphoreType.DMA],
    )
    def k(ys_hbm, d_hbm, w_hbm, out_hbm, idxv, rows, wvm, sem, sem2):
        wid = lax.axis_index("s") * _NC + lax.axis_index("c")
        base = wid * tpw
        half = tpw // 2
        pltpu.sync_copy(d_hbm.at[pl.ds(base, tpw)], idxv)
        pltpu.sync_copy(w_hbm.at[pl.ds(base, tpw)], wvm)
        cp0 = pltpu.async_copy(ys_hbm.at[idxv.at[pl.ds(0, half)]],
                               rows.at[pl.ds(0, half)], sem)
        cp1 = pltpu.async_copy(ys_hbm.at[idxv.at[pl.ds(half, half)]],
                               rows.at[pl.ds(half, half)], sem2)

        def rbody(r, carry):
            wv = wvm[pl.ds((r // 16) * 16, 16)]
            wb = wv[jnp.broadcast_to(r % 16, (16,))]
            for k2 in range(D // 16):
                sl = pl.ds(k2 * 16, 16)
                rows[r, sl] = rows[r, sl] * wb
            return carry

        cp0.wait()
        lax.fori_loop(0, half, rbody, 0)
        pltpu.sync_copy(rows.at[pl.ds(0, half)],
                        out_hbm.at[pl.ds(base, half)])
        cp1.wait()
        lax.fori_loop(half, tpw, rbody, 0)
        pltpu.sync_copy(rows.at[pl.ds(half, half)],
                        out_hbm.at[pl.ds(base + half, half)])

    return k(ys, dest, w)


FSPLIT = 1
FH = F // FSPLIT


def _ffn_body(plan_ref, xs_ref, w1_ref, b1_ref, w2_ref, b2_ref,
              ys_ref):
    i = pl.program_id(0)
    j = pl.program_id(1)
    na = plan_ref[2, 0]

    @pl.when(jnp.logical_and(i < na, j == 0))
    def _():
        ys_ref[...] = jnp.broadcast_to(b2_ref[0], (BLK, D))

    @pl.when(i < na)
    def _():
        xb = xs_ref[...].astype(jnp.bfloat16)
        h = jnp.dot(xb, w1_ref[0].astype(jnp.bfloat16),
                    preferred_element_type=jnp.float32)
        h = h + b1_ref[0]
        h = 0.5 * h * (1.0 + lax.erf(h * 0.7071067811865476))
        y = jnp.dot(h.astype(jnp.bfloat16), w2_ref[0].astype(jnp.bfloat16),
                    preferred_element_type=jnp.float32)
        ys_ref[...] += y


def _ffn(xs, w1, b1, w2, b2, plan):
    grid_spec = pltpu.PrefetchScalarGridSpec(
        num_scalar_prefetch=1,
        grid=(NBLK, FSPLIT),
        in_specs=[
            pl.BlockSpec((BLK, D), lambda i, j, p:
                         (jnp.minimum(i, p[2, 0] - 1), 0)),
            pl.BlockSpec((1, D, FH), lambda i, j, p: (p[1, i], 0, j)),
            pl.BlockSpec((1, 1, FH), lambda i, j, p: (p[1, i], 0, j)),
            pl.BlockSpec((1, FH, D), lambda i, j, p: (p[1, i], j, 0)),
            pl.BlockSpec((1, 1, D), lambda i, j, p: (p[1, i], 0, 0)),
        ],
        out_specs=pl.BlockSpec((BLK, D), lambda i, j, p: (i, 0)),
    )
    return pl.pallas_call(
        _ffn_body,
        grid_spec=grid_spec,
        out_shape=jax.ShapeDtypeStruct((TPAD, D), jnp.float32),
        compiler_params=pltpu.CompilerParams(
            dimension_semantics=("arbitrary", "arbitrary")),
    )(plan, xs, w1, b1.reshape(E, 1, F), w2, b2.reshape(E, 1, D))


def kernel(x, Wr, W1, b1, W2, b2):
    bsz, seq, _ = x.shape
    x_flat = x.reshape(T, D)
    eidx, rank, w, plan, aux = _routing(x_flat, Wr)
    xs, dest = _sc_dispatch(x_flat, eidx, rank, plan)
    ys = _ffn(xs, W1, b1, W2, b2, plan)
    outf = _sc_combine(ys, dest, w)
    return outf.reshape(bsz, seq, D), aux[0, 0]


# speedup vs baseline: 1.0206x; 1.0206x over previous
import functools

import jax
import jax.numpy as jnp
from jax import lax
from jax.experimental import pallas as pl
from jax.experimental.pallas import tpu as pltpu
from jax.experimental.pallas import tpu_sc as plsc

T = 2048
D = 768
E = 8
F = 3072
TB = 256
BLK = 384
NBLK = (T + BLK - 1) // BLK + E
TPAD = NBLK * BLK
LBW = 0.01
_NC, _NS = 2, 16
_NW = _NC * _NS


def _routing_body(x_ref, wr_ref, eidx_ref, rank_ref, w_ref, plan_ref,
                  aux_ref, cnt, sp):
    i = pl.program_id(0)

    @pl.when(i == 0)
    def _init():
        cnt[...] = jnp.zeros_like(cnt)
        sp[...] = jnp.zeros_like(sp)

    logits = jnp.dot(x_ref[...], wr_ref[...],
                     preferred_element_type=jnp.float32)
    m = jnp.max(logits, axis=-1, keepdims=True)
    ex = jnp.exp(logits - m)
    s = jnp.sum(ex, axis=-1, keepdims=True)
    probs16 = jnp.concatenate(
        [ex / s, jnp.zeros((TB, 16 - E), jnp.float32)], axis=-1)
    lane8 = lax.broadcasted_iota(jnp.int32, (TB, E), 1)
    e = jnp.min(jnp.where(logits == m, lane8, E), axis=-1).astype(jnp.int32)
    onehot = (lax.broadcasted_iota(jnp.int32, (TB, 16), 1)
              == e[:, None]).astype(jnp.float32)
    tri = (lax.broadcasted_iota(jnp.int32, (TB, TB), 0)
           >= lax.broadcasted_iota(jnp.int32, (TB, TB), 1)).astype(jnp.float32)
    csum = jnp.dot(tri, onehot, preferred_element_type=jnp.float32)
    prev = cnt[0, :]
    rank = jnp.sum(onehot * (prev[None, :] + csum), axis=-1) - 1.0
    eidx_ref[...] = e
    rank_ref[...] = rank.astype(jnp.int32)
    w_ref[...] = 1.0 / s[:, 0]
    cnt[0, :] = prev + jnp.sum(onehot, axis=0)
    sp[0, :] = sp[0, :] + jnp.sum(probs16, axis=0)

    @pl.when(i == pl.num_programs(0) - 1)
    def _fin():
        cfin = cnt[0:1, :]
        nb = jnp.floor((cfin + (BLK - 1)) * (1.0 / BLK))
        tri16 = (lax.broadcasted_iota(jnp.int32, (16, 16), 0)
                 <= lax.broadcasted_iota(jnp.int32, (16, 16), 1)
                 ).astype(jnp.float32)
        cumnb = jnp.dot(nb, tri16, preferred_element_type=jnp.float32)
        offs = (cumnb - nb) * BLK
        col = lax.broadcasted_iota(jnp.int32, (16, 16), 1)
        row = lax.broadcasted_iota(jnp.int32, (16, 16), 0)
        ble = jnp.sum(jnp.where(
            jnp.logical_and(col < E,
                            jnp.broadcast_to(cumnb, (16, 16))
                            <= row.astype(jnp.float32)),
            1, 0), axis=-1, keepdims=False)
        ble = jnp.minimum(ble, E - 1)
        nact = jnp.sum(jnp.where(col < E, jnp.broadcast_to(nb, (16, 16)), 0.0),
                       axis=-1)[0:1]
        plan_ref[...] = jnp.concatenate([
            offs.astype(jnp.int32),
            ble.astype(jnp.int32)[None, :],
            jnp.broadcast_to(nact.astype(jnp.int32)[None, :1], (1, 16)),
            jnp.zeros((5, 16), jnp.int32)], axis=0)
        aux = (LBW * E / (T * T)) * jnp.sum(sp[0, :] * cfin[0, :])
        aux_ref[...] = jnp.broadcast_to(aux, (1, 1))


def _routing(x_flat, wr):
    nb = T // TB
    return pl.pallas_call(
        _routing_body,
        grid=(nb,),
        in_specs=[
            pl.BlockSpec((TB, D), lambda i: (i, 0)),
            pl.BlockSpec((D, E), lambda i: (0, 0)),
        ],
        out_specs=[
            pl.BlockSpec((TB,), lambda i: (i,)),
            pl.BlockSpec((TB,), lambda i: (i,)),
            pl.BlockSpec((TB,), lambda i: (i,)),
            pl.BlockSpec((8, 16), lambda i: (0, 0)),
            pl.BlockSpec((1, 1), lambda i: (0, 0)),
        ],
        out_shape=[
            jax.ShapeDtypeStruct((T,), jnp.int32),
            jax.ShapeDtypeStruct((T,), jnp.int32),
            jax.ShapeDtypeStruct((T,), jnp.float32),
            jax.ShapeDtypeStruct((8, 16), jnp.int32),
            jax.ShapeDtypeStruct((1, 1), jnp.float32),
        ],
        scratch_shapes=[pltpu.VMEM((1, 16), jnp.float32),
                        pltpu.VMEM((1, 16), jnp.float32)],
        compiler_params=pltpu.CompilerParams(
            dimension_semantics=("arbitrary",)),
    )(x_flat, wr)


def _sc_dispatch(x_flat, eidx, rank, plan):
    tpw = T // _NW
    mesh = plsc.VectorSubcoreMesh(core_axis_name="c", subcore_axis_name="s")

    @functools.partial(
        pl.kernel, mesh=mesh,
        out_type=[jax.ShapeDtypeStruct((TPAD, D), jnp.float32),
                  jax.ShapeDtypeStruct((T,), jnp.int32)],
        scratch_types=[pltpu.VMEM((16,), jnp.int32),
                       pltpu.VMEM((tpw,), jnp.int32),
                       pltpu.VMEM((tpw,), jnp.int32),
                       pltpu.VMEM((tpw,), jnp.int32),
                       pltpu.VMEM((tpw, D), jnp.float32),
                       pltpu.SemaphoreType.DMA,
                       pltpu.SemaphoreType.DMA],
    )
    def k(x_hbm, e_hbm, r_hbm, p_hbm, xs_hbm, dest_hbm,
          ovm, evm, rvm, dvm, rows, sem, xsem):
        wid = lax.axis_index("s") * _NC + lax.axis_index("c")
        base = wid * tpw
        xcp = pltpu.async_copy(x_hbm.at[pl.ds(base, tpw)], rows, xsem)
        ecp = pltpu.async_copy(e_hbm.at[pl.ds(base, tpw)], evm, sem)
        rcp = pltpu.async_copy(r_hbm.at[pl.ds(base, tpw)], rvm, sem)
        pcp = pltpu.async_copy(p_hbm.at[0], ovm, sem)
        ecp.wait()
        rcp.wait()
        pcp.wait()
        offs_reg = ovm[...]
        for c in range(tpw // 16):
            sl = pl.ds(c * 16, 16)
            dvm[sl] = offs_reg[evm[sl]] + rvm[sl]
        xcp.wait()
        pltpu.async_copy(rows, xs_hbm.at[dvm], sem).wait()
        pltpu.sync_copy(dvm, dest_hbm.at[pl.ds(base, tpw)])

    return k(x_flat, eidx, rank, plan)


def _sc_combine(ys, dest, w):
    tpw = T // _NW
    mesh = plsc.VectorSubcoreMesh(core_axis_name="c", subcore_axis_name="s")

    @functools.partial(
        pl.kernel, mesh=mesh,
        out_type=jax.ShapeDtypeStruct((T, D), jnp.float32),
        scratch_types=[pltpu.VMEM((tpw,), jnp.int32),
                       pltpu.VMEM((tpw, D), jnp.float32),
                       pltpu.VMEM((tpw,), jnp.float32),
                       pltpu.SemaphoreType.DMA,
                       pltpu.SemaphoreType.DMA],
    )
    def k(ys_hbm, d_hbm, w_hbm, out_hbm, idxv, rows, wvm, sem, sem2):
        wid = lax.axis_index("s") * _NC + lax.axis_index("c")
        base = wid * tpw
        half = tpw // 2
        pltpu.sync_copy(d_hbm.at[pl.ds(base, tpw)], idxv)
        pltpu.sync_copy(w_hbm.at[pl.ds(base, tpw)], wvm)
        cp0 = pltpu.async_copy(ys_hbm.at[idxv.at[pl.ds(0, half)]],
                               rows.at[pl.ds(0, half)], sem)
        cp1 = pltpu.async_copy(ys_hbm.at[idxv.at[pl.ds(half, half)]],
                               rows.at[pl.ds(half, half)], sem2)

        def rbody(r, carry):
            wv = wvm[pl.ds((r // 16) * 16, 16)]
            wb = wv[jnp.broadcast_to(r % 16, (16,))]
            for k2 in range(D // 16):
                sl = pl.ds(k2 * 16, 16)
                rows[r, sl] = rows[r, sl] * wb
            return carry

        cp0.wait()
        lax.fori_loop(0, half, rbody, 0)
        pltpu.sync_copy(rows.at[pl.ds(0, half)],
                        out_hbm.at[pl.ds(base, half)])
        cp1.wait()
        lax.fori_loop(half, tpw, rbody, 0)
        pltpu.sync_copy(rows.at[pl.ds(half, half)],
                        out_hbm.at[pl.ds(base + half, half)])

    return k(ys, dest, w)


FSPLIT = 1
FH = F // FSPLIT


def _ffn_body(plan_ref, xs_ref, w1_ref, b1_ref, w2_ref, b2_ref,
              ys_ref):
    i = pl.program_id(0)
    j = pl.program_id(1)
    na = plan_ref[2, 0]

    @pl.when(jnp.logical_and(i < na, j == 0))
    def _():
        ys_ref[...] = jnp.broadcast_to(b2_ref[0], (BLK, D))

    @pl.when(i < na)
    def _():
        xb = xs_ref[...].astype(jnp.bfloat16)
        h = jnp.dot(xb, w1_ref[0].astype(jnp.bfloat16),
                    preferred_element_type=jnp.float32)
        h = h + b1_ref[0]
        h = 0.5 * h * (1.0 + lax.erf(h * 0.7071067811865476))
        y = jnp.dot(h.astype(jnp.bfloat16), w2_ref[0].astype(jnp.bfloat16),
                    preferred_element_type=jnp.float32)
        ys_ref[...] += y


def _ffn(xs, w1, b1, w2, b2, plan):
    grid_spec = pltpu.PrefetchScalarGridSpec(
        num_scalar_prefetch=1,
        grid=(NBLK, FSPLIT),
        in_specs=[
            pl.BlockSpec((BLK, D), lambda i, j, p:
                         (jnp.minimum(i, p[2, 0] - 1), 0)),
            pl.BlockSpec((1, D, FH), lambda i, j, p: (p[1, i], 0, j)),
            pl.BlockSpec((1, 1, FH), lambda i, j, p: (p[1, i], 0, j)),
            pl.BlockSpec((1, FH, D), lambda i, j, p: (p[1, i], j, 0)),
            pl.BlockSpec((1, 1, D), lambda i, j, p: (p[1, i], 0, 0)),
        ],
        out_specs=pl.BlockSpec((BLK, D), lambda i, j, p: (i, 0)),
    )
    return pl.pallas_call(
        _ffn_body,
        grid_spec=grid_spec,
        out_shape=jax.ShapeDtypeStruct((TPAD, D), jnp.float32),
        compiler_params=pltpu.CompilerParams(
            dimension_semantics=("arbitrary", "arbitrary")),
    )(plan, xs, w1, b1.reshape(E, 1, F), w2, b2.reshape(E, 1, D))


def kernel(x, Wr, W1, b1, W2, b2):
    bsz, seq, _ = x.shape
    x_flat = x.reshape(T, D)
    eidx, rank, w, plan, aux = _routing(x_flat, Wr)
    xs, dest = _sc_dispatch(x_flat, eidx, rank, plan)
    ys = _ffn(xs, W1, b1, W2, b2, plan)
    outf = _sc_combine(ys, dest, w)
    return outf.reshape(bsz, seq, D), aux[0, 0]
